# baseline (speedup 1.0000x reference)
import functools

import jax
import jax.numpy as jnp
from jax import lax
from jax.experimental import pallas as pl
from jax.experimental.pallas import tpu as pltpu
from jax.experimental.pallas import tpu_sc as plsc

_NC = 2
_NS = 16
_NW = _NC * _NS
_CHUNK = 128


def kernel(p_h, p_t, p_r, n_h, n_t, n_r, ent_emb, rel_emb):
    B = p_h.shape[0]
    ENT, D = ent_emb.shape
    REL = rel_emb.shape[0]
    PACK = 128 // D
    bpw = B // _NW
    n_chunks = bpw // _CHUNK

    ent2 = ent_emb.reshape(ENT // PACK, 128)
    rel_f = rel_emb.reshape(REL * D)

    mesh = plsc.VectorSubcoreMesh(
        core_axis_name="c", subcore_axis_name="s",
        num_cores=_NC, num_subcores=_NS)

    out_t = jax.ShapeDtypeStruct((B,), jnp.float32)
    scratch = (
        [pltpu.VMEM((n_chunks, _CHUNK), jnp.int32) for _ in range(6)]
        + [pltpu.VMEM((n_chunks, _CHUNK), jnp.int32) for _ in range(4)]
        + [pltpu.VMEM((_CHUNK, 128), jnp.float32) for _ in range(2)]
        + [pltpu.VMEM((REL * D,), jnp.float32)]
        + [pltpu.VMEM((_CHUNK * 16,), jnp.float32)]
        + [pltpu.VMEM((bpw,), jnp.float32) for _ in range(2)]
        + [pltpu.SemaphoreType.DMA]
    )

    @functools.partial(
        pl.kernel,
        out_type=(out_t, out_t),
        mesh=mesh,
        scratch_types=scratch,
        compiler_params=pltpu.CompilerParams(needs_layout_passes=False),
    )
    def run(ph_h, pt_h, pr_h, nh_h, nt_h, nr_h, ent_h, rel_h,
            po_h, no_h,
            iv0, iv1, iv2, iv3, iv4, iv5,
            gv0, gv1, gv2, gv3,
            hbuf, tbuf, rel_v, dred_v, op_v, on_v, sem):
        wid = lax.axis_index("s") * _NC + lax.axis_index("c")
        base = wid * bpw

        rel_cp = pltpu.async_copy(rel_h, rel_v, sem)

        idx_hbm = [ph_h, pt_h, nh_h, nt_h, pr_h, nr_h]
        idx_v = [iv0, iv1, iv2, iv3, iv4, iv5]
        for ih, iv in zip(idx_hbm, idx_v):
            for k in range(n_chunks):
                pltpu.sync_copy(ih.at[pl.ds(base + k * _CHUNK, _CHUNK)],
                                iv.at[k])

        shift = PACK.bit_length() - 1
        gidx_v = [gv0, gv1, gv2, gv3]
        for iv, gv in zip(idx_v[:4], gidx_v):
            for k in range(n_chunks):
                for v in range(_CHUNK // 16):
                    e = iv[k, pl.ds(v * 16, 16)]
                    gv[k, pl.ds(v * 16, 16)] = lax.shift_right_logical(
                        e, shift)

        rel_cp.wait()

        iota16 = lax.iota(jnp.int32, 16)
        mask_slab = PACK - 1

        def do_side(ihv, itv, irv, ghv, gtv, o_ref):
            for k in range(n_chunks):
                h_cp = pltpu.async_copy(ent_h.at[ghv.at[k]], hbuf, sem)
                t_cp = pltpu.async_copy(ent_h.at[gtv.at[k]], tbuf, sem)
                h_cp.wait()
                t_cp.wait()

                def body1(g, carry):
                    eh16 = ihv[k, pl.ds(g * 16, 16)]
                    et16 = itv[k, pl.ds(g * 16, 16)]
                    er16 = irv[k, pl.ds(g * 16, 16)]
                    sh16 = (eh16 & mask_slab) * D
                    st16 = (et16 & mask_slab) * D
                    ro16 = er16 * D
                    for j in range(16):
                        row = g * 16 + j
                        d = jnp.zeros((16,), jnp.float32)
                        for c in range(D // 16):
                            hv = hbuf[row, pl.ds(sh16[j] + c * 16, 16)]
                            tv = tbuf[row, pl.ds(st16[j] + c * 16, 16)]
                            rv = rel_v[pl.ds(ro16[j] + c * 16, 16)]
                            d = d + jnp.abs(hv + rv - tv)
                        dred_v[pl.ds(row * 16, 16)] = d
                    return carry
                lax.fori_loop(0, _CHUNK // 16, body1, 0)

                def body2(g, carry):
                    base_idx = g * 256 + iota16 * 16
                    acc = jnp.zeros((16,), jnp.float32)
                    for j in range(16):
                        acc = acc + plsc.load_gather(dred_v, [base_idx + j])
                    o_ref[pl.ds(k * _CHUNK + g * 16, 16)] = acc
                    return carry
                lax.fori_loop(0, _CHUNK // 16, body2, 0)

        do_side(iv0, iv1, iv4, gv0, gv1, op_v)
        do_side(iv2, iv3, iv5, gv2, gv3, on_v)

        pltpu.sync_copy(op_v, po_h.at[pl.ds(base, bpw)])
        pltpu.sync_copy(on_v, no_h.at[pl.ds(base, bpw)])

    return run(p_h, p_t, p_r, n_h, n_t, n_r, ent2, rel_f)
